# Initial kernel scaffold; baseline (speedup 1.0000x reference)
import functools

import jax
import jax.numpy as jnp
from jax import lax
from jax.experimental import pallas as pl
from jax.experimental.pallas import tpu as pltpu
from jax.experimental.pallas import tpu_sc as plsc

D = 64
LANES = 16
NCORES = 2
NSUB = 16
NW = NCORES * NSUB
CHUNK = 64

_PI = 3.141592653589793
_TWO_PI = 6.283185307179586
_INV_PI = 0.3183098861837907
_MAGIC = 12582912.0

_C9 = 2.7557319e-06
_C7 = -1.9841270e-04
_C5 = 8.3333333e-03
_C3 = -0.16666667


def _sin(x):
    q = x * _INV_PI
    t = q + _MAGIC
    kf = t - _MAGIC
    sgn = lax.shift_left(plsc.bitcast(t, jnp.int32), 31)
    r = x - kf * _PI
    r2 = r * r
    p = ((((_C9 * r2) + _C7) * r2 + _C5) * r2 + _C3) * r2 + 1.0
    s = r * p
    return plsc.bitcast(jnp.bitwise_xor(plsc.bitcast(s, jnp.int32), sgn), jnp.float32)


def _body(n_per_w, n_chunks,
          h_hbm, t_hbm, r_hbm, d_hbm,
          emb_E, emb_E_var, emb_R, emb_R_var, emb_TE, alpha_E, beta_E,
          omega_E, emb_TR, alpha_R, beta_R, omega_R,
          out_hbm,
          idxh_v, idxt_v, idxr_v, dv_v,
          eh_v, et_v, teh_v, tet_v, bh_v, oh_v, bt_v, ot_v, vh_v, vt_v,
          re_v, rte_v, rb_v, ro_v, rv_v,
          ah_v, at_v, ar_v, outb_v, sem):
    wid = lax.axis_index("s") * NCORES + lax.axis_index("c")
    base = wid * n_per_w

    def chunk_body(c, carry):
        off = base + c * CHUNK
        pltpu.sync_copy(h_hbm.at[pl.ds(off, CHUNK)], idxh_v)
        pltpu.sync_copy(t_hbm.at[pl.ds(off, CHUNK)], idxt_v)
        pltpu.sync_copy(r_hbm.at[pl.ds(off, CHUNK)], idxr_v)
        pltpu.sync_copy(d_hbm.at[pl.ds(off, CHUNK)], dv_v)
        cps = []
        for tab, idx, dst in (
            (emb_E, idxh_v, eh_v), (emb_E, idxt_v, et_v),
            (emb_TE, idxh_v, teh_v), (emb_TE, idxt_v, tet_v),
            (beta_E, idxh_v, bh_v), (omega_E, idxh_v, oh_v),
            (beta_E, idxt_v, bt_v), (omega_E, idxt_v, ot_v),
            (emb_E_var, idxh_v, vh_v), (emb_E_var, idxt_v, vt_v),
            (emb_R, idxr_v, re_v), (emb_TR, idxr_v, rte_v),
            (beta_R, idxr_v, rb_v), (omega_R, idxr_v, ro_v),
            (emb_R_var, idxr_v, rv_v),
            (alpha_E, idxh_v, ah_v), (alpha_E, idxt_v, at_v),
            (alpha_R, idxr_v, ar_v),
        ):
            cps.append(pltpu.async_copy(tab.at[idx], dst, sem))
        for cp in cps:
            cp.wait()

        def samp_body(si, carry2):
            d_s = dv_v[si]
            dah = d_s * ah_v[si, 0]
            dat = d_s * at_v[si, 0]
            dar = d_s * ar_v[si, 0]
            td = _TWO_PI * d_s
            acc = jnp.zeros((LANES,), jnp.float32)
            for j in range(D // LANES):
                sl = pl.ds(j * LANES, LANES)
                hm = eh_v[si, sl] + dah * teh_v[si, sl] + bh_v[si, sl] * _sin(td * oh_v[si, sl])
                tm = et_v[si, sl] + dat * tet_v[si, sl] + bt_v[si, sl] * _sin(td * ot_v[si, sl])
                rm = re_v[si, sl] + dar * rte_v[si, sl] + rb_v[si, sl] * _sin(td * ro_v[si, sl])
                m = rm + tm - hm
                sv = vh_v[si, sl] + vt_v[si, sl]
                rv = rv_v[si, sl]
                sq = m * m
                acc = acc + (sv + sq) / rv + (rv + sq) / sv
            outb_v[si] = jnp.sum(acc) * 0.25 - (D / 2.0)
            return carry2

        lax.fori_loop(0, CHUNK, samp_body, 0)
        pltpu.sync_copy(outb_v, out_hbm.at[pl.ds(off, CHUNK)])
        return carry

    lax.fori_loop(0, n_chunks, chunk_body, 0)


def kernel(sample, emb_E, emb_E_var, emb_R, emb_R_var, emb_TE, alpha_E,
           beta_E, omega_E, emb_TR, alpha_R, beta_R, omega_R):
    bs = sample.shape[0]
    s = sample.reshape(-1, 4)
    n = s.shape[0]
    assert n % (NW * CHUNK) == 0
    n_per_w = n // NW
    n_chunks = n_per_w // CHUNK
    h_i = s[:, 0].astype(jnp.int32)
    r_i = s[:, 1].astype(jnp.int32)
    t_i = s[:, 2].astype(jnp.int32)
    d_i = s[:, 3]

    mesh = plsc.VectorSubcoreMesh(core_axis_name="c", subcore_axis_name="s")
    row = pltpu.VMEM((CHUNK, D), jnp.float32)
    col = pltpu.VMEM((CHUNK, 1), jnp.float32)
    run = pl.kernel(
        functools.partial(_body, n_per_w, n_chunks),
        out_type=jax.ShapeDtypeStruct((n,), jnp.float32),
        mesh=mesh,
        scratch_types=[
            pltpu.VMEM((CHUNK,), jnp.int32),
            pltpu.VMEM((CHUNK,), jnp.int32),
            pltpu.VMEM((CHUNK,), jnp.int32),
            pltpu.VMEM((CHUNK,), jnp.float32),
            row, row, row, row, row, row, row, row, row, row,
            row, row, row, row, row,
            col, col, col,
            pltpu.VMEM((CHUNK,), jnp.float32),
            pltpu.SemaphoreType.DMA,
        ],
    )
    scores = run(h_i, t_i, r_i, d_i, emb_E, emb_E_var, emb_R, emb_R_var,
                 emb_TE, alpha_E, beta_E, omega_E, emb_TR, alpha_R, beta_R,
                 omega_R)
    return scores.reshape(bs, -1)


# speedup vs baseline: 3.6290x; 3.6290x over previous
import functools

import jax
import jax.numpy as jnp
from jax import lax
from jax.experimental import pallas as pl
from jax.experimental.pallas import tpu as pltpu
from jax.experimental.pallas import tpu_sc as plsc

D = 64
LANES = 16
NCORES = 2
NSUB = 16
NW = NCORES * NSUB
CHUNK = 64

_PI = 3.141592653589793
_TWO_PI = 6.283185307179586
_INV_PI = 0.3183098861837907
_MAGIC = 12582912.0

_C9 = 2.7557319e-06
_C7 = -1.9841270e-04
_C5 = 8.3333333e-03
_C3 = -0.16666667


def _sin(x):
    q = x * _INV_PI
    t = q + _MAGIC
    kf = t - _MAGIC
    sgn = lax.shift_left(lax.bitcast_convert_type(t, jnp.int32), 31)
    r = x - kf * _PI
    r2 = r * r
    p = ((((_C9 * r2) + _C7) * r2 + _C5) * r2 + _C3) * r2 + 1.0
    s = r * p
    return lax.bitcast_convert_type(
        jnp.bitwise_xor(lax.bitcast_convert_type(s, jnp.int32), sgn),
        jnp.float32)


def _body(n_per_w, n_chunks,
          h_hbm, t_hbm, r_hbm, d_hbm,
          emb_E, emb_E_var, emb_R, emb_R_var, emb_TE, alpha_E, beta_E,
          omega_E, emb_TR, alpha_R, beta_R, omega_R,
          out_hbm,
          idxh_v, idxt_v, idxr_v, dv_v,
          eh_v, et_v, teh_v, tet_v, bh_v, oh_v, bt_v, ot_v, vh_v, vt_v,
          re_v, rte_v, rb_v, ro_v, rv_v,
          ah_v, at_v, ar_v, outb_v, sem):
    wid = lax.axis_index("s") * NCORES + lax.axis_index("c")
    base = wid * n_per_w

    def chunk_body(c, carry):
        off = base + c * CHUNK
        pltpu.sync_copy(h_hbm.at[pl.ds(off, CHUNK)], idxh_v)
        pltpu.sync_copy(t_hbm.at[pl.ds(off, CHUNK)], idxt_v)
        pltpu.sync_copy(r_hbm.at[pl.ds(off, CHUNK)], idxr_v)
        pltpu.sync_copy(d_hbm.at[pl.ds(off, CHUNK)], dv_v.at[pl.ds(0, CHUNK)])
        cps = []
        for tab, idx, dst in (
            (emb_E, idxh_v, eh_v), (emb_E, idxt_v, et_v),
            (emb_TE, idxh_v, teh_v), (emb_TE, idxt_v, tet_v),
            (beta_E, idxh_v, bh_v), (omega_E, idxh_v, oh_v),
            (beta_E, idxt_v, bt_v), (omega_E, idxt_v, ot_v),
            (emb_E_var, idxh_v, vh_v), (emb_E_var, idxt_v, vt_v),
            (emb_R, idxr_v, re_v), (emb_TR, idxr_v, rte_v),
            (beta_R, idxr_v, rb_v), (omega_R, idxr_v, ro_v),
            (emb_R_var, idxr_v, rv_v),
            (alpha_E, idxh_v, ah_v.at[pl.ds(0, CHUNK)]),
            (alpha_E, idxt_v, at_v.at[pl.ds(0, CHUNK)]),
            (alpha_R, idxr_v, ar_v.at[pl.ds(0, CHUNK)]),
        ):
            cps.append(pltpu.async_copy(tab.at[idx], dst, sem))
        for cp in cps:
            cp.wait()

        lane_iota = lax.iota(jnp.int32, LANES)

        def group_body(g, carry2):
            gbase = g * LANES

            def lane_body(l, svec):
                si = gbase + l
                d_s = dv_v[pl.ds(si, LANES)][0]
                dah = d_s * ah_v[pl.ds(si, LANES)][0]
                dat = d_s * at_v[pl.ds(si, LANES)][0]
                dar = d_s * ar_v[pl.ds(si, LANES)][0]
                td = _TWO_PI * d_s
                acc = jnp.zeros((LANES,), jnp.float32)
                for j in range(D // LANES):
                    sl = pl.ds(j * LANES, LANES)
                    hm = eh_v[si, sl] + dah * teh_v[si, sl] + bh_v[si, sl] * _sin(td * oh_v[si, sl])
                    tm = et_v[si, sl] + dat * tet_v[si, sl] + bt_v[si, sl] * _sin(td * ot_v[si, sl])
                    rm = re_v[si, sl] + dar * rte_v[si, sl] + rb_v[si, sl] * _sin(td * ro_v[si, sl])
                    m = rm + tm - hm
                    sv = vh_v[si, sl] + vt_v[si, sl]
                    rv = rv_v[si, sl]
                    sq = m * m
                    acc = acc + (sv + sq) / rv + (rv + sq) / sv
                tot = jnp.sum(acc) * 0.25 - (D / 2.0)
                return jnp.where(lane_iota == l, tot, svec)

            svec = lax.fori_loop(0, LANES, lane_body,
                                 jnp.zeros((LANES,), jnp.float32))
            outb_v[pl.ds(gbase, LANES)] = svec
            return carry2

        lax.fori_loop(0, CHUNK // LANES, group_body, 0)
        pltpu.sync_copy(outb_v, out_hbm.at[pl.ds(off, CHUNK)])
        return carry

    lax.fori_loop(0, n_chunks, chunk_body, 0)


def kernel(sample, emb_E, emb_E_var, emb_R, emb_R_var, emb_TE, alpha_E,
           beta_E, omega_E, emb_TR, alpha_R, beta_R, omega_R):
    bs = sample.shape[0]
    s = sample.reshape(-1, 4)
    n = s.shape[0]
    assert n % (NW * CHUNK) == 0
    n_per_w = n // NW
    n_chunks = n_per_w // CHUNK
    h_i = s[:, 0].astype(jnp.int32)
    r_i = s[:, 1].astype(jnp.int32)
    t_i = s[:, 2].astype(jnp.int32)
    d_i = s[:, 3]
    alpha_E = alpha_E.reshape(-1)
    alpha_R = alpha_R.reshape(-1)

    mesh = plsc.VectorSubcoreMesh(core_axis_name="c", subcore_axis_name="s")
    row = pltpu.VMEM((CHUNK, D), jnp.float32)
    col = pltpu.VMEM((CHUNK + LANES,), jnp.float32)
    run = pl.kernel(
        functools.partial(_body, n_per_w, n_chunks),
        out_type=jax.ShapeDtypeStruct((n,), jnp.float32),
        mesh=mesh,
        compiler_params=pltpu.CompilerParams(needs_layout_passes=False,
                                             use_tc_tiling_on_sc=False),
        scratch_types=[
            pltpu.VMEM((CHUNK,), jnp.int32),
            pltpu.VMEM((CHUNK,), jnp.int32),
            pltpu.VMEM((CHUNK,), jnp.int32),
            col,
            row, row, row, row, row, row, row, row, row, row,
            row, row, row, row, row,
            col, col, col,
            pltpu.VMEM((CHUNK,), jnp.float32),
            pltpu.SemaphoreType.DMA,
        ],
    )
    scores = run(h_i, t_i, r_i, d_i, emb_E, emb_E_var, emb_R, emb_R_var,
                 emb_TE, alpha_E, beta_E, omega_E, emb_TR, alpha_R, beta_R,
                 omega_R)
    return scores.reshape(bs, -1)


# speedup vs baseline: 3.9891x; 1.0992x over previous
import functools

import jax
import jax.numpy as jnp
from jax import lax
from jax.experimental import pallas as pl
from jax.experimental.pallas import tpu as pltpu
from jax.experimental.pallas import tpu_sc as plsc

D = 64
W = 2 * D
LANES = 16
NCORES = 2
NSUB = 16
NW = NCORES * NSUB
CHUNK = 64

_PI = 3.141592653589793
_TWO_PI = 6.283185307179586
_INV_PI = 0.3183098861837907
_MAGIC = 12582912.0

_C9 = 2.7557319e-06
_C7 = -1.9841270e-04
_C5 = 8.3333333e-03
_C3 = -0.16666667


def _sin(x):
    q = x * _INV_PI
    t = q + _MAGIC
    kf = t - _MAGIC
    sgn = lax.shift_left(lax.bitcast_convert_type(t, jnp.int32), 31)
    r = x - kf * _PI
    r2 = r * r
    p = ((((_C9 * r2) + _C7) * r2 + _C5) * r2 + _C3) * r2 + 1.0
    s = r * p
    return lax.bitcast_convert_type(
        jnp.bitwise_xor(lax.bitcast_convert_type(s, jnp.int32), sgn),
        jnp.float32)


def _body(n_per_w, n_chunks,
          h_hbm, t_hbm, r_hbm, d_hbm,
          et_t, bo_t, va_t, rt_t, rbo_t, rva_t,
          out_hbm,
          idxh_s, idxt_s, idxr_s, dv_s,
          eth_v, boh_v, vah_v, ett_v, bot_v, vat_v, rt_v, rbo_v, rva_v,
          outb_v, sem):
    wid = lax.axis_index("s") * NCORES + lax.axis_index("c")
    base = wid * n_per_w
    pltpu.sync_copy(h_hbm.at[pl.ds(base, n_per_w)], idxh_s)
    pltpu.sync_copy(t_hbm.at[pl.ds(base, n_per_w)], idxt_s)
    pltpu.sync_copy(r_hbm.at[pl.ds(base, n_per_w)], idxr_s)
    pltpu.sync_copy(d_hbm.at[pl.ds(base, n_per_w)],
                    dv_s.at[pl.ds(0, n_per_w)])
    lane_iota = lax.iota(jnp.int32, LANES)

    def chunk_body(c, carry):
        loc = c * CHUNK
        cps = []
        for tab, idx, dst in (
            (et_t, idxh_s, eth_v), (bo_t, idxh_s, boh_v), (va_t, idxh_s, vah_v),
            (et_t, idxt_s, ett_v), (bo_t, idxt_s, bot_v), (va_t, idxt_s, vat_v),
            (rt_t, idxr_s, rt_v), (rbo_t, idxr_s, rbo_v), (rva_t, idxr_s, rva_v),
        ):
            cps.append(pltpu.async_copy(tab.at[idx.at[pl.ds(loc, CHUNK)]],
                                        dst, sem))
        for cp in cps:
            cp.wait()

        def group_body(g, carry2):
            gbase = g * LANES

            def lane_body(l, svec):
                si = gbase + l
                d_s = dv_s[pl.ds(loc + si, LANES)][0]
                dah = d_s * vah_v[si, pl.ds(D, LANES)][0]
                dat = d_s * vat_v[si, pl.ds(D, LANES)][0]
                dar = d_s * rva_v[si, pl.ds(D, LANES)][0]
                td = _TWO_PI * d_s
                acc = jnp.zeros((LANES,), jnp.float32)
                for j in range(D // LANES):
                    lo = pl.ds(j * LANES, LANES)
                    hi = pl.ds(D + j * LANES, LANES)
                    hm = (eth_v[si, lo] + dah * eth_v[si, hi]
                          + boh_v[si, lo] * _sin(td * boh_v[si, hi]))
                    tm = (ett_v[si, lo] + dat * ett_v[si, hi]
                          + bot_v[si, lo] * _sin(td * bot_v[si, hi]))
                    rm = (rt_v[si, lo] + dar * rt_v[si, hi]
                          + rbo_v[si, lo] * _sin(td * rbo_v[si, hi]))
                    m = rm + tm - hm
                    sv = vah_v[si, lo] + vat_v[si, lo]
                    rv = rva_v[si, lo]
                    sq = m * m
                    acc = acc + (sv + sq) / rv + (rv + sq) / sv
                tot = jnp.sum(acc) * 0.25 - (D / 2.0)
                return jnp.where(lane_iota == l, tot, svec)

            svec = lax.fori_loop(0, LANES, lane_body,
                                 jnp.zeros((LANES,), jnp.float32))
            outb_v[pl.ds(gbase, LANES)] = svec
            return carry2

        lax.fori_loop(0, CHUNK // LANES, group_body, 0)
        pltpu.sync_copy(outb_v, out_hbm.at[pl.ds(base + loc, CHUNK)])
        return carry

    lax.fori_loop(0, n_chunks, chunk_body, 0)


def kernel(sample, emb_E, emb_E_var, emb_R, emb_R_var, emb_TE, alpha_E,
           beta_E, omega_E, emb_TR, alpha_R, beta_R, omega_R):
    bs = sample.shape[0]
    s = sample.reshape(-1, 4)
    n = s.shape[0]
    assert n % (NW * CHUNK) == 0
    n_per_w = n // NW
    n_chunks = n_per_w // CHUNK
    h_i = s[:, 0].astype(jnp.int32)
    r_i = s[:, 1].astype(jnp.int32)
    t_i = s[:, 2].astype(jnp.int32)
    d_i = s[:, 3]

    ne = emb_E.shape[0]
    nr = emb_R.shape[0]
    padE = jnp.zeros((ne, D - 1), jnp.float32)
    padR = jnp.zeros((nr, D - 1), jnp.float32)
    et_t = jnp.concatenate([emb_E, emb_TE], axis=1)
    bo_t = jnp.concatenate([beta_E, omega_E], axis=1)
    va_t = jnp.concatenate([emb_E_var, alpha_E, padE], axis=1)
    rt_t = jnp.concatenate([emb_R, emb_TR], axis=1)
    rbo_t = jnp.concatenate([beta_R, omega_R], axis=1)
    rva_t = jnp.concatenate([emb_R_var, alpha_R, padR], axis=1)

    mesh = plsc.VectorSubcoreMesh(core_axis_name="c", subcore_axis_name="s")
    row = pltpu.VMEM((CHUNK, W), jnp.float32)
    stg = pltpu.VMEM((n_per_w,), jnp.int32)
    run = pl.kernel(
        functools.partial(_body, n_per_w, n_chunks),
        out_type=jax.ShapeDtypeStruct((n,), jnp.float32),
        mesh=mesh,
        compiler_params=pltpu.CompilerParams(needs_layout_passes=False,
                                             use_tc_tiling_on_sc=True),
        scratch_types=[
            stg, stg, stg,
            pltpu.VMEM((n_per_w + LANES,), jnp.float32),
            row, row, row, row, row, row, row, row, row,
            pltpu.VMEM((CHUNK,), jnp.float32),
            pltpu.SemaphoreType.DMA,
        ],
    )
    scores = run(h_i, t_i, r_i, d_i, et_t, bo_t, va_t, rt_t, rbo_t, rva_t)
    return scores.reshape(bs, -1)
